# baseline (speedup 1.0000x reference)
import functools

import jax
import jax.numpy as jnp
from jax import lax
from jax.experimental import pallas as pl
from jax.experimental.pallas import tpu as pltpu
from jax.experimental.pallas import tpu_sc as plsc

_BATCH = 16384
_NUM_CORES = 2
_NUM_SUBCORES = 16
_NUM_WORKERS = _NUM_CORES * _NUM_SUBCORES
_CHUNK = _BATCH // _NUM_WORKERS
_HALF = _CHUNK // 2
_LANES = 16
_SCALE = 1.702 * 4.0 * 8.0


def _gather_theta_body(user_id, theta_t, out, idx_u, tv, sem_i, sem_t):
    wid = lax.axis_index("s") * _NUM_CORES + lax.axis_index("c")
    base = wid * _CHUNK
    pltpu.async_copy(user_id.at[pl.ds(base, _CHUNK)], idx_u, sem_i).wait()
    pltpu.async_copy(theta_t.at[idx_u], tv, sem_t).wait()
    pltpu.sync_copy(tv, out.at[pl.ds(base, _CHUNK)])


def _irt_body(question_id, theta_vals, a_t, b_t, c_t, out,
              idx_q, tv, av, bv, cv, ov,
              sem_iq, sem_tv,
              s_a0, s_b0, s_c0, s_a1, s_b1, s_c1):
    wid = lax.axis_index("s") * _NUM_CORES + lax.axis_index("c")
    base = wid * _CHUNK

    ct = pltpu.async_copy(theta_vals.at[pl.ds(base, _CHUNK)], tv, sem_tv)
    cq = pltpu.async_copy(question_id.at[pl.ds(base, _CHUNK)], idx_q, sem_iq)
    cq.wait()

    sems = ((s_a0, s_b0, s_c0), (s_a1, s_b1, s_c1))
    cps = []
    for h in range(2):
        off = h * _HALF
        iq = idx_q.at[pl.ds(off, _HALF)]
        sa, sb, sc = sems[h]
        cps.append((
            pltpu.async_copy(a_t.at[iq], av.at[pl.ds(off, _HALF)], sa),
            pltpu.async_copy(b_t.at[iq], bv.at[pl.ds(off, _HALF)], sb),
            pltpu.async_copy(c_t.at[iq], cv.at[pl.ds(off, _HALF)], sc),
        ))
    ct.wait()

    for h in range(2):
        for cp in cps[h]:
            cp.wait()

        def step(i, _, _h=h):
            off = pl.multiple_of(_h * _HALF + i * _LANES, _LANES)
            et = jnp.exp(-tv[pl.ds(off, _LANES)])
            ea = jnp.exp(-av[pl.ds(off, _LANES)])
            eb = jnp.exp(-bv[pl.ds(off, _LANES)])
            ec = jnp.exp(-cv[pl.ds(off, _LANES)])
            z = _SCALE * (eb - et) / ((1.0 + ea) * (1.0 + et) * (1.0 + eb))
            ez = jnp.exp(-z)
            ov[pl.ds(off, _LANES)] = (1.0 + ez + ec) / ((1.0 + ez) * (1.0 + ec))
            return 0

        lax.fori_loop(0, _HALF // _LANES, step, 0, unroll=2)

    pltpu.sync_copy(ov, out.at[pl.ds(base, _CHUNK)])


@jax.jit
def _irt_sc(user_id, question_id, theta_table, a_table, b_table, c_table):
    mesh = plsc.VectorSubcoreMesh(core_axis_name="c", subcore_axis_name="s")

    theta_t = theta_table.reshape(-1)
    gather_theta = functools.partial(
        pl.kernel,
        mesh=mesh,
        out_type=jax.ShapeDtypeStruct((_BATCH,), jnp.float32),
        scratch_types=[
            pltpu.VMEM((_CHUNK,), jnp.int32),
            pltpu.VMEM((_CHUNK,), jnp.float32),
            pltpu.SemaphoreType.DMA,
            pltpu.SemaphoreType.DMA,
        ],
    )(_gather_theta_body)
    theta_vals = gather_theta(user_id, theta_t)

    a_t = a_table.reshape(-1)
    b_t = b_table.reshape(-1)
    c_t = c_table.reshape(-1)
    irt = functools.partial(
        pl.kernel,
        mesh=mesh,
        out_type=jax.ShapeDtypeStruct((_BATCH,), jnp.float32),
        scratch_types=[
            pltpu.VMEM((_CHUNK,), jnp.int32),
            pltpu.VMEM((_CHUNK,), jnp.float32),
            pltpu.VMEM((_CHUNK,), jnp.float32),
            pltpu.VMEM((_CHUNK,), jnp.float32),
            pltpu.VMEM((_CHUNK,), jnp.float32),
            pltpu.VMEM((_CHUNK,), jnp.float32),
        ] + [pltpu.SemaphoreType.DMA] * 8,
    )(_irt_body)
    return irt(question_id, theta_vals, a_t, b_t, c_t)


def kernel(user_id, question_id, theta_table, a_table, b_table, c_table):
    return _irt_sc(
        user_id.astype(jnp.int32),
        question_id.astype(jnp.int32),
        theta_table,
        a_table,
        b_table,
        c_table,
    )


# speedup vs baseline: 1.0286x; 1.0286x over previous
import functools

import jax
import jax.numpy as jnp
from jax import lax
from jax.experimental import pallas as pl
from jax.experimental.pallas import tpu as pltpu
from jax.experimental.pallas import tpu_sc as plsc

_BATCH = 16384
_NUM_CORES = 2
_NUM_SUBCORES = 16
_NUM_WORKERS = _NUM_CORES * _NUM_SUBCORES
_CHUNK = _BATCH // _NUM_WORKERS
_NQ = 4
_QS = _CHUNK // _NQ
_LANES = 16
_SCALE = 1.702 * 4.0 * 8.0


def _irt_body(user_id, question_id, theta_t, a_t, b_t, c_t, out,
              idx_u, idx_q, tv, av, bv, cv, ov,
              sem_iu, sem_iq, sem_out, *gsems):
    wid = lax.axis_index("s") * _NUM_CORES + lax.axis_index("c")
    base = wid * _CHUNK

    cu = pltpu.async_copy(user_id.at[pl.ds(base, _CHUNK)], idx_u, sem_iu)
    cq = pltpu.async_copy(question_id.at[pl.ds(base, _CHUNK)], idx_q, sem_iq)
    cu.wait()
    cq.wait()

    cps = []
    for q in range(_NQ):
        off = q * _QS
        iu = idx_u.at[pl.ds(off, _QS)]
        iq = idx_q.at[pl.ds(off, _QS)]
        st, sa, sb, sc = gsems[4 * q:4 * q + 4]
        cps.append((
            pltpu.async_copy(theta_t.at[iu], tv.at[pl.ds(off, _QS)], st),
            pltpu.async_copy(a_t.at[iq], av.at[pl.ds(off, _QS)], sa),
            pltpu.async_copy(b_t.at[iq], bv.at[pl.ds(off, _QS)], sb),
            pltpu.async_copy(c_t.at[iq], cv.at[pl.ds(off, _QS)], sc),
        ))

    outs = []
    for q in range(_NQ):
        for cp in cps[q]:
            cp.wait()

        def step(i, _, _q=q):
            off = pl.multiple_of(_q * _QS + i * _LANES, _LANES)
            et = jnp.exp(-tv[pl.ds(off, _LANES)])
            ea = jnp.exp(-av[pl.ds(off, _LANES)])
            eb = jnp.exp(-bv[pl.ds(off, _LANES)])
            ec = jnp.exp(-cv[pl.ds(off, _LANES)])
            z = _SCALE * (eb - et) / ((1.0 + ea) * (1.0 + et) * (1.0 + eb))
            ez = jnp.exp(-z)
            ov[pl.ds(off, _LANES)] = (1.0 + ez + ec) / ((1.0 + ez) * (1.0 + ec))
            return 0

        lax.fori_loop(0, _QS // _LANES, step, 0, unroll=2)
        off = q * _QS
        outs.append(pltpu.async_copy(
            ov.at[pl.ds(off, _QS)], out.at[pl.ds(base + off, _QS)], sem_out))

    for cp in outs:
        cp.wait()


@jax.jit
def _irt_sc(user_id, question_id, theta_t, a_t, b_t, c_t):
    mesh = plsc.VectorSubcoreMesh(core_axis_name="c", subcore_axis_name="s")
    f = functools.partial(
        pl.kernel,
        mesh=mesh,
        out_type=jax.ShapeDtypeStruct((_BATCH,), jnp.float32),
        scratch_types=[
            pltpu.VMEM((_CHUNK,), jnp.int32),
            pltpu.VMEM((_CHUNK,), jnp.int32),
            pltpu.VMEM((_CHUNK,), jnp.float32),
            pltpu.VMEM((_CHUNK,), jnp.float32),
            pltpu.VMEM((_CHUNK,), jnp.float32),
            pltpu.VMEM((_CHUNK,), jnp.float32),
            pltpu.VMEM((_CHUNK,), jnp.float32),
        ] + [pltpu.SemaphoreType.DMA] * (3 + 4 * _NQ),
    )(_irt_body)
    return f(user_id, question_id, theta_t, a_t, b_t, c_t)


def kernel(user_id, question_id, theta_table, a_table, b_table, c_table):
    return _irt_sc(
        user_id.astype(jnp.int32),
        question_id.astype(jnp.int32),
        theta_table.reshape(-1),
        a_table.reshape(-1),
        b_table.reshape(-1),
        c_table.reshape(-1),
    )


# speedup vs baseline: 1.0288x; 1.0003x over previous
import functools

import jax
import jax.numpy as jnp
from jax import lax
from jax.experimental import pallas as pl
from jax.experimental.pallas import tpu as pltpu
from jax.experimental.pallas import tpu_sc as plsc

_BATCH = 16384
_NUM_CORES = 2
_NUM_SUBCORES = 16
_NUM_WORKERS = _NUM_CORES * _NUM_SUBCORES
_CHUNK = _BATCH // _NUM_WORKERS
_HALF = _CHUNK // 2
_LANES = 16
_SCALE = 1.702 * 4.0 * 8.0


def _irt_body(user_id, question_id, theta_t, a_t, b_t, c_t, out,
              idx_u, idx_q, tv, av, bv, cv, ov,
              sem_iu, sem_iq, sem_out,
              s_t0, s_a0, s_b0, s_c0, s_t1, s_a1, s_b1, s_c1):
    wid = lax.axis_index("s") * _NUM_CORES + lax.axis_index("c")
    base = wid * _CHUNK

    cu = pltpu.async_copy(user_id.at[pl.ds(base, _CHUNK)], idx_u, sem_iu)
    cq = pltpu.async_copy(question_id.at[pl.ds(base, _CHUNK)], idx_q, sem_iq)
    cu.wait()
    cq.wait()

    sems = ((s_t0, s_a0, s_b0, s_c0), (s_t1, s_a1, s_b1, s_c1))
    cps = []
    for h in range(2):
        off = h * _HALF
        iu = idx_u.at[pl.ds(off, _HALF)]
        iq = idx_q.at[pl.ds(off, _HALF)]
        st, sa, sb, sc = sems[h]
        cps.append((
            pltpu.async_copy(theta_t.at[iu], tv.at[pl.ds(off, _HALF)], st),
            pltpu.async_copy(a_t.at[iq], av.at[pl.ds(off, _HALF)], sa),
            pltpu.async_copy(b_t.at[iq], bv.at[pl.ds(off, _HALF)], sb),
            pltpu.async_copy(c_t.at[iq], cv.at[pl.ds(off, _HALF)], sc),
        ))

    outs = []
    for h in range(2):
        for cp in cps[h]:
            cp.wait()

        def step(i, _, _h=h):
            off = pl.multiple_of(_h * _HALF + i * _LANES, _LANES)
            et = jnp.exp(-tv[pl.ds(off, _LANES)])
            ea = jnp.exp(-av[pl.ds(off, _LANES)])
            eb = jnp.exp(-bv[pl.ds(off, _LANES)])
            ec = jnp.exp(-cv[pl.ds(off, _LANES)])
            z = _SCALE * (eb - et) / ((1.0 + ea) * (1.0 + et) * (1.0 + eb))
            ez = jnp.exp(-z)
            ov[pl.ds(off, _LANES)] = (1.0 + ez + ec) / ((1.0 + ez) * (1.0 + ec))
            return 0

        lax.fori_loop(0, _HALF // _LANES, step, 0, unroll=2)
        off = h * _HALF
        outs.append(pltpu.async_copy(
            ov.at[pl.ds(off, _HALF)], out.at[pl.ds(base + off, _HALF)], sem_out))

    for cp in outs:
        cp.wait()


@jax.jit
def _irt_sc(user_id, question_id, theta_t, a_t, b_t, c_t):
    mesh = plsc.VectorSubcoreMesh(core_axis_name="c", subcore_axis_name="s")
    f = functools.partial(
        pl.kernel,
        mesh=mesh,
        out_type=jax.ShapeDtypeStruct((_BATCH,), jnp.float32),
        scratch_types=[
            pltpu.VMEM((_CHUNK,), jnp.int32),
            pltpu.VMEM((_CHUNK,), jnp.int32),
            pltpu.VMEM((_CHUNK,), jnp.float32),
            pltpu.VMEM((_CHUNK,), jnp.float32),
            pltpu.VMEM((_CHUNK,), jnp.float32),
            pltpu.VMEM((_CHUNK,), jnp.float32),
            pltpu.VMEM((_CHUNK,), jnp.float32),
        ] + [pltpu.SemaphoreType.DMA] * 11,
    )(_irt_body)
    return f(user_id, question_id, theta_t, a_t, b_t, c_t)


def kernel(user_id, question_id, theta_table, a_table, b_table, c_table):
    return _irt_sc(
        user_id.astype(jnp.int32),
        question_id.astype(jnp.int32),
        theta_table.reshape(-1),
        a_table.reshape(-1),
        b_table.reshape(-1),
        c_table.reshape(-1),
    )


# speedup vs baseline: 1.0529x; 1.0234x over previous
import functools

import jax
import jax.numpy as jnp
from jax import lax
from jax.experimental import pallas as pl
from jax.experimental.pallas import tpu as pltpu
from jax.experimental.pallas import tpu_sc as plsc

_BATCH = 16384
_NUM_CORES = 2
_NUM_SUBCORES = 16
_NUM_WORKERS = _NUM_CORES * _NUM_SUBCORES
_CHUNK = _BATCH // _NUM_WORKERS
_HALF = _CHUNK // 2
_LANES = 16
_SCALE = 1.702 * 4.0 * 8.0


def _irt_body(user_id, question_id, theta_t, a_t, b_t, c_t, out,
              idx_u, idx_q, tv, av, bv, cv, ov,
              sem_iu, sem_iq, sem_out,
              s_t0, s_a0, s_b0, s_c0, s_t1, s_a1, s_b1, s_c1):
    wid = lax.axis_index("s") * _NUM_CORES + lax.axis_index("c")
    base = wid * _CHUNK

    cu = pltpu.async_copy(user_id.at[pl.ds(base, _CHUNK)], idx_u, sem_iu)
    cq = pltpu.async_copy(question_id.at[pl.ds(base, _CHUNK)], idx_q, sem_iq)
    cu.wait()
    cq.wait()

    sems = ((s_t0, s_a0, s_b0, s_c0), (s_t1, s_a1, s_b1, s_c1))
    cps = []
    for h in range(2):
        off = h * _HALF
        iu = idx_u.at[pl.ds(off, _HALF)]
        iq = idx_q.at[pl.ds(off, _HALF)]
        st, sa, sb, sc = sems[h]
        cps.append((
            pltpu.async_copy(theta_t.at[iu], tv.at[pl.ds(off, _HALF)], st),
            pltpu.async_copy(a_t.at[iq], av.at[pl.ds(off, _HALF)], sa),
            pltpu.async_copy(b_t.at[iq], bv.at[pl.ds(off, _HALF)], sb),
            pltpu.async_copy(c_t.at[iq], cv.at[pl.ds(off, _HALF)], sc),
        ))

    outs = []
    for h in range(2):
        for cp in cps[h]:
            cp.wait()

        def step(i, _, _h=h):
            off = pl.multiple_of(_h * _HALF + i * _LANES, _LANES)
            et = jnp.exp(-tv[pl.ds(off, _LANES)])
            ea = jnp.exp(-av[pl.ds(off, _LANES)])
            eb = jnp.exp(-bv[pl.ds(off, _LANES)])
            ec = jnp.exp(-cv[pl.ds(off, _LANES)])
            z = _SCALE * (eb - et) / ((1.0 + ea) * (1.0 + et) * (1.0 + eb))
            ez = jnp.exp(-z)
            ov[pl.ds(off, _LANES)] = (1.0 + ez + ec) / ((1.0 + ez) * (1.0 + ec))
            return 0

        lax.fori_loop(0, _HALF // _LANES, step, 0, unroll=1)
        off = h * _HALF
        outs.append(pltpu.async_copy(
            ov.at[pl.ds(off, _HALF)], out.at[pl.ds(base + off, _HALF)], sem_out))

    for cp in outs:
        cp.wait()


@jax.jit
def _irt_sc(user_id, question_id, theta_t, a_t, b_t, c_t):
    mesh = plsc.VectorSubcoreMesh(core_axis_name="c", subcore_axis_name="s")
    f = functools.partial(
        pl.kernel,
        mesh=mesh,
        out_type=jax.ShapeDtypeStruct((_BATCH,), jnp.float32),
        scratch_types=[
            pltpu.VMEM((_CHUNK,), jnp.int32),
            pltpu.VMEM((_CHUNK,), jnp.int32),
            pltpu.VMEM((_CHUNK,), jnp.float32),
            pltpu.VMEM((_CHUNK,), jnp.float32),
            pltpu.VMEM((_CHUNK,), jnp.float32),
            pltpu.VMEM((_CHUNK,), jnp.float32),
            pltpu.VMEM((_CHUNK,), jnp.float32),
        ] + [pltpu.SemaphoreType.DMA] * 11,
    )(_irt_body)
    return f(user_id, question_id, theta_t, a_t, b_t, c_t)


def kernel(user_id, question_id, theta_table, a_table, b_table, c_table):
    return _irt_sc(
        user_id.astype(jnp.int32),
        question_id.astype(jnp.int32),
        theta_table.reshape(-1),
        a_table.reshape(-1),
        b_table.reshape(-1),
        c_table.reshape(-1),
    )
